# baseline (speedup 1.0000x reference)
import jax
import jax.numpy as jnp
from jax.experimental import pallas as pl
from jax.experimental.pallas import tpu as pltpu


def _add_kernel(x_ref, pos_ref, o_ref):
    o_ref[...] = x_ref[...] + pos_ref[...]


def _half(x, pos_table, b0, nb):
    batch, seq_len, d_model = x.shape
    s_blk = 256
    grid = (seq_len // s_blk,)
    return pl.pallas_call(
        _add_kernel,
        grid=grid,
        in_specs=[
            pl.BlockSpec((nb, s_blk, d_model), lambda s: (b0 // nb, s, 0)),
            pl.BlockSpec((s_blk, d_model), lambda s: (s, 0)),
        ],
        out_specs=pl.BlockSpec((nb, s_blk, d_model), lambda s: (0, s, 0)),
        out_shape=jax.ShapeDtypeStruct((nb, seq_len, d_model), x.dtype),
    )(x, pos_table)


def kernel(x, pos_table):
    lo = _half(x, pos_table, 0, 2)
    hi = _half(x, pos_table, 2, 2)
    return jnp.concatenate([lo, hi], axis=0)
